# Initial kernel scaffold; baseline (speedup 1.0000x reference)
import jax
import jax.numpy as jnp
from jax.experimental import pallas as pl


def kernel(node_features, neighbor_idx, weights, W1, b1, W2, b2, W3, b3, gamma, beta):
    raise NotImplementedError("write your pallas kernel here")


# speedup vs baseline: 39.6745x; 39.6745x over previous
import functools

import jax
import jax.numpy as jnp
from jax import lax
from jax.experimental import pallas as pl
from jax.experimental.pallas import tpu as pltpu
from jax.experimental.pallas import tpu_sc as plsc

B, N, K, D, M = 2, 10000, 16, 128, 32
BN = B * N
TOTAL_IDX = BN * K

NSUB = 16
PER_W = (N * K) // NSUB
CHUNK = 80
NCHUNK = PER_W // CHUNK
NPC = CHUNK // K
FPC = NPC * M
PAIRS = (NCHUNK - 1) // 2


def _gelu(x):
    return 0.5 * x * (1.0 + lax.erf(x * 0.7071067811865476))


def _encode_body(nf_ref, idx_ref, w1_ref, b1_ref, q_ref, fidx_ref, *, rows):
    x = nf_ref[...]
    q = jnp.dot(x, w1_ref[...], preferred_element_type=jnp.float32) + b1_ref[...]
    q_ref[...] = jnp.concatenate(
        [_gelu(q), jnp.zeros((rows, D - M), jnp.float32)], axis=1)
    fidx_ref[...] = jnp.maximum(idx_ref[...], 0)


def _encode(nf_flat, idx_flat, W1, b1):
    rows = 2000
    grid = (BN // rows,)
    return pl.pallas_call(
        functools.partial(_encode_body, rows=rows),
        grid=grid,
        in_specs=[
            pl.BlockSpec((rows, D), lambda i: (i, 0)),
            pl.BlockSpec((rows, K), lambda i: (i, 0)),
            pl.BlockSpec((D, M), lambda i: (0, 0)),
            pl.BlockSpec((1, M), lambda i: (0, 0)),
        ],
        out_specs=[
            pl.BlockSpec((rows, D), lambda i: (i, 0)),
            pl.BlockSpec((rows, K), lambda i: (i, 0)),
        ],
        out_shape=[
            jax.ShapeDtypeStruct((BN, D), jnp.float32),
            jax.ShapeDtypeStruct((BN, K), jnp.int32),
        ],
    )(nf_flat, idx_flat, W1, b1)


def _sc_gather_sum(qpad, idx4, w3d):
    mesh = plsc.VectorSubcoreMesh(core_axis_name="c", subcore_axis_name="s")

    @functools.partial(
        pl.kernel,
        mesh=mesh,
        out_type=jax.ShapeDtypeStruct((BN * M,), jnp.float32),
        scratch_types=[
            pltpu.VMEM_SHARED((N, D), jnp.float32),
            pltpu.VMEM((NCHUNK, CHUNK), jnp.int32),
            pltpu.VMEM((PER_W,), jnp.float32),
            pltpu.VMEM((2, CHUNK, D), jnp.float32),
            pltpu.VMEM((2 * FPC,), jnp.float32),
            pltpu.SemaphoreType.DMA,
            pltpu.SemaphoreType.DMA,
        ],
    )
    def gather_kernel(q_hbm, idx_hbm, w_hbm, out_hbm,
                      tab, idx_v, w_v, rows_v, acc_v, sem_a, sem_b):
        c = lax.axis_index("c")
        s = lax.axis_index("s")
        rps = N // 2

        @pl.when(s < 2)
        def _stage():
            off = pl.multiple_of(c * N + s * rps, 8)
            pltpu.sync_copy(q_hbm.at[pl.ds(off, rps)],
                            tab.at[pl.ds(s * rps, rps)])

        pltpu.sync_copy(idx_hbm.at[c, s], idx_v)
        pltpu.sync_copy(w_hbm.at[c, s], w_v)
        plsc.subcore_barrier()

        base = (c * NSUB + s) * (PER_W * 2)

        def fire(ch, bb, sem):
            pltpu.async_copy(tab.at[idx_v.at[ch]], rows_v.at[bb], sem)

        def wait(bb, sem):
            pltpu.make_async_copy(q_hbm.at[pl.ds(0, CHUNK)],
                                  rows_v.at[bb], sem).wait()

        def compute(ch, bb):
            for n in range(NPC):
                e0 = n * K
                woff = pl.multiple_of(ch * CHUNK, 8) + e0
                wv16 = w_v[pl.ds(woff, K)]
                acc0 = jnp.zeros((16,), jnp.float32)
                acc1 = jnp.zeros((16,), jnp.float32)
                for k in range(K):
                    w = wv16[k]
                    acc0 = acc0 + w * rows_v[bb, e0 + k, pl.ds(0, 16)]
                    acc1 = acc1 + w * rows_v[bb, e0 + k, pl.ds(16, 16)]
                acc_v[pl.ds(bb * FPC + n * M, 16)] = acc0
                acc_v[pl.ds(bb * FPC + n * M + 16, 16)] = acc1

        fire(0, 0, sem_a)
        fire(1, 1, sem_b)

        @pl.loop(0, PAIRS)
        def _pair(it):
            ch0 = it * 2
            wait(0, sem_a)
            compute(ch0, 0)

            @pl.when(ch0 + 2 < NCHUNK)
            def _f0():
                fire(ch0 + 2, 0, sem_a)

            wait(1, sem_b)
            compute(ch0 + 1, 1)

            @pl.when(ch0 + 3 < NCHUNK)
            def _f1():
                fire(ch0 + 3, 1, sem_b)

            off = pl.multiple_of(base + it * (2 * FPC), 8)
            pltpu.sync_copy(acc_v, out_hbm.at[pl.ds(off, 2 * FPC)])

        wait(0, sem_a)
        compute(NCHUNK - 1, 0)
        off_e = pl.multiple_of(base + (NCHUNK - 1) * FPC, 8)
        pltpu.sync_copy(acc_v.at[pl.ds(0, FPC)], out_hbm.at[pl.ds(off_e, FPC)])

    return gather_kernel(qpad, idx4, w3d)


def _update_body(nf_ref, s_ref, w_ref, w2_ref, b2_ref, w3_ref, b3_ref,
                 gamma_ref, beta_ref, o_ref):
    w = w_ref[...]
    sw = jnp.sum(w, axis=1, keepdims=True)
    agg = jnp.dot(s_ref[...], w2_ref[...], preferred_element_type=jnp.float32) \
        + sw * b2_ref[...]
    nf = nf_ref[...]
    u = (jnp.dot(nf, w3_ref[0:D, :], preferred_element_type=jnp.float32)
         + jnp.dot(agg, w3_ref[D:D + M, :], preferred_element_type=jnp.float32)
         + b3_ref[...])
    mean = jnp.mean(u, axis=1, keepdims=True)
    cen = u - mean
    var = jnp.mean(cen * cen, axis=1, keepdims=True)
    ln = cen * lax.rsqrt(var + 1e-5) * gamma_ref[...] + beta_ref[...]
    o_ref[...] = _gelu(ln)


def _update(nf_flat, s2, w_flat, W2, b2, W3, b3, gamma, beta):
    rows = 2000
    grid = (BN // rows,)
    return pl.pallas_call(
        _update_body,
        grid=grid,
        in_specs=[
            pl.BlockSpec((rows, D), lambda i: (i, 0)),
            pl.BlockSpec((rows, M), lambda i: (i, 0)),
            pl.BlockSpec((rows, K), lambda i: (i, 0)),
            pl.BlockSpec((M, M), lambda i: (0, 0)),
            pl.BlockSpec((1, M), lambda i: (0, 0)),
            pl.BlockSpec((D + M, D), lambda i: (0, 0)),
            pl.BlockSpec((1, D), lambda i: (0, 0)),
            pl.BlockSpec((1, D), lambda i: (0, 0)),
            pl.BlockSpec((1, D), lambda i: (0, 0)),
        ],
        out_specs=pl.BlockSpec((rows, D), lambda i: (i, 0)),
        out_shape=jax.ShapeDtypeStruct((BN, D), jnp.float32),
    )(nf_flat, s2, w_flat, W2, b2, W3, b3, gamma, beta)


def kernel(node_features, neighbor_idx, weights, W1, b1, W2, b2, W3, b3,
           gamma, beta):
    nf_flat = node_features.reshape(BN, D)
    idx_flat = neighbor_idx.reshape(BN, K)

    qpad, fidx = _encode(nf_flat, idx_flat, W1, b1.reshape(1, M))
    idx4 = fidx.reshape(B, NSUB, NCHUNK, CHUNK)
    w3d = weights.reshape(B, NSUB, PER_W)
    s1d = _sc_gather_sum(qpad, idx4, w3d)
    s2 = s1d.reshape(BN, M)
    out = _update(nf_flat, s2, weights.reshape(BN, K), W2, b2.reshape(1, M),
                  W3, b3.reshape(1, D), gamma.reshape(1, D), beta.reshape(1, D))
    return out.reshape(B, N, D)


# speedup vs baseline: 41.0607x; 1.0349x over previous
import functools

import jax
import jax.numpy as jnp
from jax import lax
from jax.experimental import pallas as pl
from jax.experimental.pallas import tpu as pltpu
from jax.experimental.pallas import tpu_sc as plsc

B, N, K, D, M = 2, 10000, 16, 128, 32
BN = B * N

NSUB = 16
NPS = N // NSUB
PER_W = NPS * K
CHUNK = 80
NROW = CHUNK // K
NCHUNK = PER_W // CHUNK
FPC = NROW * M
PAIRS = (NCHUNK - 1) // 2
SLAB = NPS + 7


def _gelu(x):
    return 0.5 * x * (1.0 + lax.erf(x * 0.7071067811865476))


def _encode_body(nf_ref, w1_ref, b1_ref, q_ref, *, rows):
    x = nf_ref[...]
    q = jnp.dot(x, w1_ref[...], preferred_element_type=jnp.float32) + b1_ref[...]
    q_ref[...] = jnp.concatenate(
        [_gelu(q), jnp.zeros((rows, D - M), jnp.float32)], axis=1)


def _encode(nf_flat, W1, b1):
    rows = 2000
    grid = (BN // rows,)
    return pl.pallas_call(
        functools.partial(_encode_body, rows=rows),
        grid=grid,
        in_specs=[
            pl.BlockSpec((rows, D), lambda i: (i, 0)),
            pl.BlockSpec((D, M), lambda i: (0, 0)),
            pl.BlockSpec((1, M), lambda i: (0, 0)),
        ],
        out_specs=pl.BlockSpec((rows, D), lambda i: (i, 0)),
        out_shape=jax.ShapeDtypeStruct((BN, D), jnp.float32),
    )(nf_flat, W1, b1)


def _sc_gather_sum(qpad, idx2d, w2d):
    mesh = plsc.VectorSubcoreMesh(core_axis_name="c", subcore_axis_name="s")

    @functools.partial(
        pl.kernel,
        mesh=mesh,
        out_type=jax.ShapeDtypeStruct((BN * M,), jnp.float32),
        scratch_types=[
            pltpu.VMEM_SHARED((N, D), jnp.float32),
            pltpu.VMEM((PER_W,), jnp.int32),
            pltpu.VMEM((PER_W,), jnp.float32),
            pltpu.VMEM((2, CHUNK, D), jnp.float32),
            pltpu.VMEM((2 * FPC,), jnp.float32),
            pltpu.SemaphoreType.DMA,
            pltpu.SemaphoreType.DMA,
        ],
    )
    def gather_kernel(q_hbm, idx_hbm, w_hbm, out_hbm,
                      tab, idx_v, w_v, rows_v, acc_v, sem_a, sem_b):
        c = lax.axis_index("c")
        s = lax.axis_index("s")
        rps = N // 2

        @pl.when(s < 2)
        def _stage():
            off = pl.multiple_of(c * N + s * rps, 8)
            pltpu.sync_copy(q_hbm.at[pl.ds(off, rps)],
                            tab.at[pl.ds(s * rps, rps)])

        wid = c * NSUB + s
        pltpu.sync_copy(idx_hbm.at[wid], idx_v)
        pltpu.sync_copy(w_hbm.at[wid], w_v)
        plsc.subcore_barrier()

        base = wid * (PER_W * 2)

        def fire(ch, bb, sem):
            pltpu.async_copy(tab.at[idx_v.at[pl.ds(ch * CHUNK, CHUNK)]],
                             rows_v.at[bb], sem)

        def wait(bb, sem):
            pltpu.make_async_copy(q_hbm.at[pl.ds(0, CHUNK)],
                                  rows_v.at[bb], sem).wait()

        def compute(ch, bb):
            for n in range(NROW):
                e0 = n * K
                wv16 = w_v[pl.ds(pl.multiple_of(ch * CHUNK, 8) + e0, K)]
                acc0 = jnp.zeros((16,), jnp.float32)
                acc1 = jnp.zeros((16,), jnp.float32)
                for k in range(K):
                    w = wv16[k]
                    acc0 = acc0 + w * rows_v[bb, e0 + k, pl.ds(0, 16)]
                    acc1 = acc1 + w * rows_v[bb, e0 + k, pl.ds(16, 16)]
                acc_v[pl.ds(bb * FPC + n * M, 16)] = acc0
                acc_v[pl.ds(bb * FPC + n * M + 16, 16)] = acc1

        fire(0, 0, sem_a)
        fire(1, 1, sem_b)

        @pl.loop(0, PAIRS)
        def _pair(it):
            ch0 = it * 2
            wait(0, sem_a)
            compute(ch0, 0)

            @pl.when(ch0 + 2 < NCHUNK)
            def _f0():
                fire(ch0 + 2, 0, sem_a)

            wait(1, sem_b)
            compute(ch0 + 1, 1)

            @pl.when(ch0 + 3 < NCHUNK)
            def _f1():
                fire(ch0 + 3, 1, sem_b)

            off = pl.multiple_of(base + it * (2 * FPC), 8)
            pltpu.sync_copy(acc_v, out_hbm.at[pl.ds(off, 2 * FPC)])

        wait(0, sem_a)
        compute(NCHUNK - 1, 0)
        off_e = pl.multiple_of(base + (NCHUNK - 1) * FPC, 8)
        pltpu.sync_copy(acc_v.at[pl.ds(0, FPC)], out_hbm.at[pl.ds(off_e, FPC)])

    return gather_kernel(qpad, idx2d, w2d)


def _update_body(nf_ref, s_ref, w_ref, w2_ref, b2_ref, w3_ref, b3_ref,
                 gamma_ref, beta_ref, o_ref, *, rows):
    w = w_ref[...]
    sw = jnp.sum(w, axis=1, keepdims=True)
    agg = jnp.dot(s_ref[...], w2_ref[...], preferred_element_type=jnp.float32) \
        + sw * b2_ref[...]
    nf = nf_ref[...]
    u = (jnp.dot(nf, w3_ref[0:D, :], preferred_element_type=jnp.float32)
         + jnp.dot(agg, w3_ref[D:D + M, :], preferred_element_type=jnp.float32)
         + b3_ref[...])
    mean = jnp.mean(u, axis=1, keepdims=True)
    cen = u - mean
    var = jnp.mean(cen * cen, axis=1, keepdims=True)
    ln = cen * lax.rsqrt(var + 1e-5) * gamma_ref[...] + beta_ref[...]
    o_ref[...] = _gelu(ln)


def _update(nf_flat, s2, w2d, W2, b2, W3, b3, gamma, beta):
    rows = 4000
    grid = (BN // rows,)
    return pl.pallas_call(
        functools.partial(_update_body, rows=rows),
        grid=grid,
        in_specs=[
            pl.BlockSpec((rows, D), lambda i: (i, 0)),
            pl.BlockSpec((rows, M), lambda i: (i, 0)),
            pl.BlockSpec((rows, K), lambda i: (i, 0)),
            pl.BlockSpec((M, M), lambda i: (0, 0)),
            pl.BlockSpec((1, M), lambda i: (0, 0)),
            pl.BlockSpec((D + M, D), lambda i: (0, 0)),
            pl.BlockSpec((1, D), lambda i: (0, 0)),
            pl.BlockSpec((1, D), lambda i: (0, 0)),
            pl.BlockSpec((1, D), lambda i: (0, 0)),
        ],
        out_specs=pl.BlockSpec((rows, D), lambda i: (i, 0)),
        out_shape=jax.ShapeDtypeStruct((BN, D), jnp.float32),
    )(nf_flat, s2, w2d, W2, b2, W3, b3, gamma, beta)


def kernel(node_features, neighbor_idx, weights, W1, b1, W2, b2, W3, b3,
           gamma, beta):
    nf_flat = node_features.reshape(BN, D)
    idx2d = neighbor_idx.reshape(BN, K)
    w2d = weights.reshape(BN, K)

    qpad = _encode(nf_flat, W1, b1.reshape(1, M))
    idx32 = idx2d.reshape(2 * NSUB, PER_W)
    w32 = w2d.reshape(2 * NSUB, PER_W)
    s1d = _sc_gather_sum(qpad, idx32, w32)
    s2 = s1d.reshape(BN, M)
    out = _update(nf_flat, s2, w2d, W2, b2.reshape(1, M),
                  W3, b3.reshape(1, D), gamma.reshape(1, D), beta.reshape(1, D))
    return out.reshape(B, N, D)


# speedup vs baseline: 48.3809x; 1.1783x over previous
import functools

import jax
import jax.numpy as jnp
from jax import lax
from jax.experimental import pallas as pl
from jax.experimental.pallas import tpu as pltpu
from jax.experimental.pallas import tpu_sc as plsc

B, N, K, D, M = 2, 10000, 16, 128, 32
BN = B * N

NSUB = 16
NPS = 624
NC = 8
CHUNK = NC * K


def _gelu(x):
    return 0.5 * x * (1.0 + lax.erf(x * 0.7071067811865476))


def _encode_body(nf_ref, w1_ref, b1_ref, q_ref, *, rows):
    x = nf_ref[...]
    q = jnp.dot(x, w1_ref[...], preferred_element_type=jnp.float32) + b1_ref[...]
    q_ref[...] = jnp.concatenate(
        [_gelu(q), jnp.zeros((rows, D - M), jnp.float32)], axis=1)


def _encode(nf_flat, W1, b1):
    rows = 2000
    grid = (BN // rows,)
    return pl.pallas_call(
        functools.partial(_encode_body, rows=rows),
        grid=grid,
        in_specs=[
            pl.BlockSpec((rows, D), lambda i: (i, 0)),
            pl.BlockSpec((D, M), lambda i: (0, 0)),
            pl.BlockSpec((1, M), lambda i: (0, 0)),
        ],
        out_specs=pl.BlockSpec((rows, D), lambda i: (i, 0)),
        out_shape=jax.ShapeDtypeStruct((BN, D), jnp.float32),
    )(nf_flat, W1, b1)


def _sc_gather_sum(qpad, idx2d, w2d):
    mesh = plsc.VectorSubcoreMesh(core_axis_name="c", subcore_axis_name="s")

    @functools.partial(
        pl.kernel,
        mesh=mesh,
        out_type=jax.ShapeDtypeStruct((BN, D), jnp.float32),
        scratch_types=[
            pltpu.VMEM_SHARED((N, D), jnp.float32),
            pltpu.VMEM((2, NC, K), jnp.int32),
            pltpu.VMEM((2, NC, K), jnp.float32),
            pltpu.VMEM((2, CHUNK), jnp.int32),
            pltpu.VMEM((2, CHUNK, D), jnp.float32),
            pltpu.VMEM((2, NC, D), jnp.float32),
            pltpu.SemaphoreType.DMA,
            pltpu.SemaphoreType.DMA,
            pltpu.SemaphoreType.DMA,
            pltpu.SemaphoreType.DMA,
            pltpu.SemaphoreType.DMA,
            pltpu.SemaphoreType.DMA,
        ],
    )
    def gather_kernel(q_hbm, idx_hbm, w_hbm, out_hbm, tab, ibuf, wbuf, f128,
                      rows_v, acc_v, g0, g1, i0, i1, w0, w1):
        c = lax.axis_index("c")
        s = lax.axis_index("s")
        gsem = (g0, g1)
        isem = (i0, i1)
        wsem = (w0, w1)
        rps = N // 2

        @pl.when(s < 2)
        def _stage():
            off = pl.multiple_of(c * N + s * rps, 8)
            pltpu.sync_copy(q_hbm.at[pl.ds(off, rps)],
                            tab.at[pl.ds(s * rps, rps)])

        nch = jnp.where(s == NSUB - 1, 80, 78)
        gb = c * N + s * NPS

        def row0(ch):
            return pl.multiple_of(gb, 8) + ch * NC

        def fire_i(ch, bb):
            pltpu.async_copy(idx_hbm.at[pl.ds(row0(ch), NC)], ibuf.at[bb],
                             isem[bb])

        def fire_w(ch, bb):
            pltpu.async_copy(w_hbm.at[pl.ds(row0(ch), NC)], wbuf.at[bb],
                             wsem[bb])

        def wait_i(bb):
            pltpu.make_async_copy(idx_hbm.at[pl.ds(0, NC)], ibuf.at[bb],
                                  isem[bb]).wait()

        def wait_w(bb):
            pltpu.make_async_copy(w_hbm.at[pl.ds(0, NC)], wbuf.at[bb],
                                  wsem[bb]).wait()

        def flat_fire_g(bb):
            for i in range(NC):
                f128[bb, pl.ds(i * K, K)] = ibuf[bb, i]
            pltpu.async_copy(tab.at[f128.at[bb]], rows_v.at[bb], gsem[bb])

        def wait_g(bb):
            pltpu.make_async_copy(q_hbm.at[pl.ds(0, CHUNK)], rows_v.at[bb],
                                  gsem[bb]).wait()

        def compute_wb(ch, bb):
            for n in range(NC):
                e0 = n * K
                wv16 = wbuf[bb, n]
                acc0 = jnp.zeros((16,), jnp.float32)
                acc1 = jnp.zeros((16,), jnp.float32)
                for k in range(K):
                    w = wv16[k]
                    acc0 = acc0 + w * rows_v[bb, e0 + k, pl.ds(0, 16)]
                    acc1 = acc1 + w * rows_v[bb, e0 + k, pl.ds(16, 16)]
                acc_v[bb, n, pl.ds(0, 16)] = acc0
                acc_v[bb, n, pl.ds(16, 16)] = acc1
            pltpu.sync_copy(acc_v.at[bb], out_hbm.at[pl.ds(row0(ch), NC)])

        fire_i(0, 0)
        fire_i(1, 1)
        fire_w(0, 0)
        fire_w(1, 1)
        plsc.subcore_barrier()
        wait_i(0)
        flat_fire_g(0)
        fire_i(2, 0)
        wait_i(1)
        flat_fire_g(1)
        fire_i(3, 1)

        @pl.loop(0, 40)
        def _pair(it):
            @pl.when(it * 2 < nch)
            def _body():
                ch0 = it * 2
                for bb in range(2):
                    ch = ch0 + bb
                    wait_g(bb)
                    wait_w(bb)
                    compute_wb(ch, bb)

                    @pl.when(ch + 2 < nch)
                    def _next():
                        fire_w(ch + 2, bb)
                        wait_i(bb)
                        flat_fire_g(bb)

                        @pl.when(ch + 4 < nch)
                        def _pref():
                            fire_i(ch + 4, bb)

    return gather_kernel(qpad, idx2d, w2d)


def _update_body(nf_ref, s_ref, w_ref, w2_ref, b2_ref, w3_ref, b3_ref,
                 gamma_ref, beta_ref, o_ref, *, rows):
    w = w_ref[...]
    sw = jnp.sum(w, axis=1, keepdims=True)
    s = s_ref[...][:, 0:M]
    agg = jnp.dot(s, w2_ref[...], preferred_element_type=jnp.float32) \
        + sw * b2_ref[...]
    nf = nf_ref[...]
    u = (jnp.dot(nf, w3_ref[0:D, :], preferred_element_type=jnp.float32)
         + jnp.dot(agg, w3_ref[D:D + M, :], preferred_element_type=jnp.float32)
         + b3_ref[...])
    mean = jnp.mean(u, axis=1, keepdims=True)
    cen = u - mean
    var = jnp.mean(cen * cen, axis=1, keepdims=True)
    ln = cen * lax.rsqrt(var + 1e-5) * gamma_ref[...] + beta_ref[...]
    o_ref[...] = _gelu(ln)


def _update(nf_flat, s2, w2d, W2, b2, W3, b3, gamma, beta):
    rows = 4000
    grid = (BN // rows,)
    return pl.pallas_call(
        functools.partial(_update_body, rows=rows),
        grid=grid,
        in_specs=[
            pl.BlockSpec((rows, D), lambda i: (i, 0)),
            pl.BlockSpec((rows, D), lambda i: (i, 0)),
            pl.BlockSpec((rows, K), lambda i: (i, 0)),
            pl.BlockSpec((M, M), lambda i: (0, 0)),
            pl.BlockSpec((1, M), lambda i: (0, 0)),
            pl.BlockSpec((D + M, D), lambda i: (0, 0)),
            pl.BlockSpec((1, D), lambda i: (0, 0)),
            pl.BlockSpec((1, D), lambda i: (0, 0)),
            pl.BlockSpec((1, D), lambda i: (0, 0)),
        ],
        out_specs=pl.BlockSpec((rows, D), lambda i: (i, 0)),
        out_shape=jax.ShapeDtypeStruct((BN, D), jnp.float32),
    )(nf_flat, s2, w2d, W2, b2, W3, b3, gamma, beta)


def kernel(node_features, neighbor_idx, weights, W1, b1, W2, b2, W3, b3,
           gamma, beta):
    nf_flat = node_features.reshape(BN, D)
    idx2d = neighbor_idx.reshape(BN, K)
    w2d = weights.reshape(BN, K)

    qpad = _encode(nf_flat, W1, b1.reshape(1, M))
    s_pad = _sc_gather_sum(qpad, idx2d, w2d)
    out = _update(nf_flat, s_pad, w2d, W2, b2.reshape(1, M),
                  W3, b3.reshape(1, D), gamma.reshape(1, D), beta.reshape(1, D))
    return out.reshape(B, N, D)


# speedup vs baseline: 48.8674x; 1.0101x over previous
import functools

import jax
import jax.numpy as jnp
from jax import lax
from jax.experimental import pallas as pl
from jax.experimental.pallas import tpu as pltpu
from jax.experimental.pallas import tpu_sc as plsc

B, N, K, D, M = 2, 10000, 16, 128, 32
BN = B * N

NSUB = 16
NPS = 624
NC = 8
CHUNK = NC * K


def _gelu(x):
    return 0.5 * x * (1.0 + lax.erf(x * 0.7071067811865476))


def _encode_body(nf_ref, w1_ref, b1_ref, q_ref, *, rows):
    x = nf_ref[...]
    q = jnp.dot(x, w1_ref[...], preferred_element_type=jnp.float32) + b1_ref[...]
    q_ref[...] = jnp.concatenate(
        [_gelu(q), jnp.zeros((rows, D - M), jnp.float32)], axis=1)


def _encode(nf_flat, W1, b1):
    rows = 2000
    grid = (BN // rows,)
    return pl.pallas_call(
        functools.partial(_encode_body, rows=rows),
        grid=grid,
        in_specs=[
            pl.BlockSpec((rows, D), lambda i: (i, 0)),
            pl.BlockSpec((D, M), lambda i: (0, 0)),
            pl.BlockSpec((1, M), lambda i: (0, 0)),
        ],
        out_specs=pl.BlockSpec((rows, D), lambda i: (i, 0)),
        out_shape=jax.ShapeDtypeStruct((BN, D), jnp.float32),
    )(nf_flat, W1, b1)


def _sc_gather_sum(qpad, idx2d, w2d):
    mesh = plsc.VectorSubcoreMesh(core_axis_name="c", subcore_axis_name="s")

    @functools.partial(
        pl.kernel,
        mesh=mesh,
        out_type=jax.ShapeDtypeStruct((BN, D), jnp.float32),
        scratch_types=[
            pltpu.VMEM_SHARED((N, D), jnp.float32),
            pltpu.VMEM((2, NC, K), jnp.int32),
            pltpu.VMEM((2, NC, K), jnp.float32),
            pltpu.VMEM((2, CHUNK), jnp.int32),
            pltpu.VMEM((2, CHUNK, D), jnp.float32),
            pltpu.VMEM((2, NC, D), jnp.float32),
            pltpu.SemaphoreType.DMA,
            pltpu.SemaphoreType.DMA,
            pltpu.SemaphoreType.DMA,
            pltpu.SemaphoreType.DMA,
            pltpu.SemaphoreType.DMA,
            pltpu.SemaphoreType.DMA,
        ],
    )
    def gather_kernel(q_hbm, idx_hbm, w_hbm, out_hbm, tab, ibuf, wbuf, f128,
                      rows_v, acc_v, g0, g1, i0, i1, w0, w1):
        c = lax.axis_index("c")
        s = lax.axis_index("s")
        gsem = (g0, g1)
        isem = (i0, i1)
        wsem = (w0, w1)
        rps = N // 2

        @pl.when(s < 2)
        def _stage():
            off = pl.multiple_of(c * N + s * rps, 8)
            pltpu.sync_copy(q_hbm.at[pl.ds(off, rps)],
                            tab.at[pl.ds(s * rps, rps)])

        nch = jnp.where(s == NSUB - 1, 80, 78)
        nb = s * NPS

        def row0(ch):
            return pl.multiple_of(nb, 8) + ch * NC

        def fire_i(ch, bb):
            pltpu.async_copy(idx_hbm.at[c, pl.ds(row0(ch), NC)], ibuf.at[bb],
                             isem[bb])

        def fire_w(ch, bb):
            pltpu.async_copy(w_hbm.at[c, pl.ds(row0(ch), NC)], wbuf.at[bb],
                             wsem[bb])

        def wait_i(bb):
            pltpu.make_async_copy(idx_hbm.at[0, pl.ds(0, NC)], ibuf.at[bb],
                                  isem[bb]).wait()

        def wait_w(bb):
            pltpu.make_async_copy(w_hbm.at[0, pl.ds(0, NC)], wbuf.at[bb],
                                  wsem[bb]).wait()

        def flat_fire_g(bb):
            for i in range(NC):
                f128[bb, pl.ds(i * K, K)] = ibuf[bb, i]
            pltpu.async_copy(tab.at[f128.at[bb]], rows_v.at[bb], gsem[bb])

        def wait_g(bb):
            pltpu.make_async_copy(q_hbm.at[pl.ds(0, CHUNK)], rows_v.at[bb],
                                  gsem[bb]).wait()

        def compute_wb(ch, bb):
            for n in range(NC):
                e0 = n * K
                wv16 = wbuf[bb, n]
                acc0 = jnp.zeros((16,), jnp.float32)
                acc1 = jnp.zeros((16,), jnp.float32)
                for k in range(K):
                    w = wv16[k]
                    acc0 = acc0 + w * rows_v[bb, e0 + k, pl.ds(0, 16)]
                    acc1 = acc1 + w * rows_v[bb, e0 + k, pl.ds(16, 16)]
                acc_v[bb, n, pl.ds(0, 16)] = acc0
                acc_v[bb, n, pl.ds(16, 16)] = acc1
            pltpu.sync_copy(
                acc_v.at[bb],
                out_hbm.at[pl.ds(pl.multiple_of(c * N, 8) + row0(ch), NC)])

        fire_i(0, 0)
        fire_i(1, 1)
        fire_w(0, 0)
        fire_w(1, 1)
        plsc.subcore_barrier()
        wait_i(0)
        flat_fire_g(0)
        fire_i(2, 0)
        wait_i(1)
        flat_fire_g(1)
        fire_i(3, 1)

        @pl.loop(0, 40)
        def _pair(it):
            @pl.when(it * 2 < nch)
            def _body():
                ch0 = it * 2
                for bb in range(2):
                    ch = ch0 + bb
                    wait_g(bb)
                    wait_w(bb)
                    compute_wb(ch, bb)

                    @pl.when(ch + 2 < nch)
                    def _next():
                        fire_w(ch + 2, bb)
                        wait_i(bb)
                        flat_fire_g(bb)

                        @pl.when(ch + 4 < nch)
                        def _pref():
                            fire_i(ch + 4, bb)

    return gather_kernel(qpad, idx2d, w2d)


def _update_body(nf_ref, s_ref, w_ref, w2_ref, b2_ref, w3_ref, b3_ref,
                 gamma_ref, beta_ref, o_ref, *, rows):
    w = w_ref[0]
    sw = jnp.sum(w, axis=1, keepdims=True)
    s = s_ref[...][:, 0:M]
    agg = jnp.dot(s, w2_ref[...], preferred_element_type=jnp.float32) \
        + sw * b2_ref[...]
    nf = nf_ref[...]
    u = (jnp.dot(nf, w3_ref[0:D, :], preferred_element_type=jnp.float32)
         + jnp.dot(agg, w3_ref[D:D + M, :], preferred_element_type=jnp.float32)
         + b3_ref[...])
    mean = jnp.mean(u, axis=1, keepdims=True)
    cen = u - mean
    var = jnp.mean(cen * cen, axis=1, keepdims=True)
    ln = cen * lax.rsqrt(var + 1e-5) * gamma_ref[...] + beta_ref[...]
    o_ref[...] = _gelu(ln)


def _update(nf_flat, s2, w3d, W2, b2, W3, b3, gamma, beta):
    rows = 5000
    bpb = N // rows
    grid = (BN // rows,)
    return pl.pallas_call(
        functools.partial(_update_body, rows=rows),
        grid=grid,
        in_specs=[
            pl.BlockSpec((rows, D), lambda i: (i, 0)),
            pl.BlockSpec((rows, D), lambda i: (i, 0)),
            pl.BlockSpec((1, rows, K), lambda i: (i // bpb, i % bpb, 0)),
            pl.BlockSpec((M, M), lambda i: (0, 0)),
            pl.BlockSpec((1, M), lambda i: (0, 0)),
            pl.BlockSpec((D + M, D), lambda i: (0, 0)),
            pl.BlockSpec((1, D), lambda i: (0, 0)),
            pl.BlockSpec((1, D), lambda i: (0, 0)),
            pl.BlockSpec((1, D), lambda i: (0, 0)),
        ],
        out_specs=pl.BlockSpec((rows, D), lambda i: (i, 0)),
        out_shape=jax.ShapeDtypeStruct((BN, D), jnp.float32),
    )(nf_flat, s2, w3d, W2, b2, W3, b3, gamma, beta)


def kernel(node_features, neighbor_idx, weights, W1, b1, W2, b2, W3, b3,
           gamma, beta):
    nf_flat = node_features.reshape(BN, D)

    qpad = _encode(nf_flat, W1, b1.reshape(1, M))
    s_pad = _sc_gather_sum(qpad, neighbor_idx, weights)
    out = _update(nf_flat, s_pad, weights, W2, b2.reshape(1, M),
                  W3, b3.reshape(1, D), gamma.reshape(1, D), beta.reshape(1, D))
    return out.reshape(B, N, D)


# speedup vs baseline: 54.6067x; 1.1174x over previous
import dataclasses
import functools

import jax
import jax.numpy as jnp
from jax import lax
from jax.experimental import pallas as pl
from jax.experimental.pallas import tpu as pltpu
from jax.experimental.pallas import tpu_sc as plsc

B, N, K, D, M = 2, 10000, 16, 128, 32
BN = B * N

NSUB = 16
NPS = 624
NC = 4
CHUNK = NC * K
WWIN = 768


def _gelu(x):
    return 0.5 * x * (1.0 + lax.erf(x * 0.7071067811865476))


def _encode_body(nf_ref, w1_ref, b1_ref, q_ref, *, rows):
    x = nf_ref[...]
    q = jnp.dot(x, w1_ref[...], preferred_element_type=jnp.float32) + b1_ref[...]
    q_ref[...] = jnp.concatenate(
        [_gelu(q), jnp.zeros((rows, D - M), jnp.float32)], axis=1)


def _encode(nf_flat, W1, b1):
    rows = 4000
    grid = (BN // rows,)
    return pl.pallas_call(
        functools.partial(_encode_body, rows=rows),
        grid=grid,
        in_specs=[
            pl.BlockSpec((rows, D), lambda i: (i, 0)),
            pl.BlockSpec((D, M), lambda i: (0, 0)),
            pl.BlockSpec((1, M), lambda i: (0, 0)),
        ],
        out_specs=pl.BlockSpec((rows, D), lambda i: (i, 0)),
        out_shape=jax.ShapeDtypeStruct((BN, D), jnp.float32),
    )(nf_flat, W1, b1)


def _sc_gather_sum(qpad, idx2d, w2d):
    mesh = plsc.VectorSubcoreMesh(core_axis_name="c", subcore_axis_name="s")
    cp = pltpu.CompilerParams()
    if "needs_layout_passes" in pltpu.CompilerParams.__dataclass_fields__:
        cp = dataclasses.replace(cp, needs_layout_passes=False)

    @functools.partial(
        pl.kernel,
        mesh=mesh,
        compiler_params=cp,
        out_type=jax.ShapeDtypeStruct((BN, D), jnp.float32),
        scratch_types=[
            pltpu.VMEM_SHARED((N, D), jnp.float32),
            pltpu.VMEM((K, WWIN), jnp.int32),
            pltpu.VMEM((K, WWIN), jnp.float32),
            pltpu.VMEM((2, CHUNK), jnp.int32),
            pltpu.VMEM((2, CHUNK, D), jnp.float32),
            pltpu.VMEM((2 * NC, D), jnp.float32),
            pltpu.SemaphoreType.DMA,
            pltpu.SemaphoreType.DMA,
        ],
    )
    def gather_kernel(q_hbm, idx_hbm, w_hbm, out_hbm, tab, iwin, wwin, f64,
                      rows_v, acc_v, g0, g1):
        c = lax.axis_index("c")
        s = lax.axis_index("s")
        gsem = (g0, g1)
        rps = N // 2

        @pl.when(s < 2)
        def _stage():
            off = pl.multiple_of(c * N + s * rps, 8)
            pltpu.sync_copy(q_hbm.at[pl.ds(off, rps)],
                            tab.at[pl.ds(s * rps, rps)])

        nb = s * NPS
        c0 = pl.multiple_of((nb // 128) * 128, 128)
        r0 = nb - c0
        pltpu.sync_copy(idx_hbm.at[c, :, pl.ds(c0, WWIN)], iwin)
        pltpu.sync_copy(w_hbm.at[c, :, pl.ds(c0, WWIN)], wwin)
        plsc.subcore_barrier()

        nch = jnp.where(s == NSUB - 1, 160, 156)
        kio = lax.iota(jnp.int32, 16)

        def flat_fire_g(ch, bb):
            for n in range(NC):
                col = jnp.full((16,), r0 + ch * NC + n, jnp.int32)
                f64[bb, pl.ds(n * K, K)] = plsc.load_gather(iwin, [kio, col])
            pltpu.async_copy(tab.at[f64.at[bb]], rows_v.at[bb], gsem[bb])

        def wait_g(bb):
            pltpu.make_async_copy(q_hbm.at[pl.ds(0, CHUNK)], rows_v.at[bb],
                                  gsem[bb]).wait()

        def compute(ch, bb):
            for n in range(NC):
                e0 = n * K
                col = jnp.full((16,), r0 + ch * NC + n, jnp.int32)
                wv16 = plsc.load_gather(wwin, [kio, col])
                acc0 = jnp.zeros((16,), jnp.float32)
                acc1 = jnp.zeros((16,), jnp.float32)
                for k in range(K):
                    w = wv16[k]
                    acc0 = acc0 + w * rows_v[bb, e0 + k, pl.ds(0, 16)]
                    acc1 = acc1 + w * rows_v[bb, e0 + k, pl.ds(16, 16)]
                acc_v[bb * NC + n, pl.ds(0, 16)] = acc0
                acc_v[bb * NC + n, pl.ds(16, 16)] = acc1
                acc_v[bb * NC + n, pl.ds(32, 16)] = jnp.full(
                    (16,), jnp.sum(wv16), jnp.float32)

        flat_fire_g(0, 0)
        flat_fire_g(1, 1)

        @pl.loop(0, 80)
        def _pair(it):
            @pl.when(it * 2 < nch)
            def _body():
                ch0 = it * 2
                for bb in range(2):
                    ch = ch0 + bb
                    wait_g(bb)
                    compute(ch, bb)

                    @pl.when(ch + 2 < nch)
                    def _next():
                        flat_fire_g(ch + 2, bb)

                row = pl.multiple_of(c * N + nb, 8) + ch0 * NC
                pltpu.sync_copy(acc_v, out_hbm.at[pl.ds(row, 2 * NC)])

    return gather_kernel(qpad, idx2d, w2d)


def _update_body(nf_ref, s_ref, w2_ref, b2_ref, w3_ref, b3_ref,
                 gamma_ref, beta_ref, o_ref, *, rows):
    spad = s_ref[...]
    s = spad[:, 0:M]
    sw = spad[:, M:M + 1]
    agg = jnp.dot(s, w2_ref[...], preferred_element_type=jnp.float32) \
        + sw * b2_ref[...]
    nf = nf_ref[...]
    u = (jnp.dot(nf, w3_ref[0:D, :], preferred_element_type=jnp.float32)
         + jnp.dot(agg, w3_ref[D:D + M, :], preferred_element_type=jnp.float32)
         + b3_ref[...])
    mean = jnp.mean(u, axis=1, keepdims=True)
    cen = u - mean
    var = jnp.mean(cen * cen, axis=1, keepdims=True)
    ln = cen * lax.rsqrt(var + 1e-5) * gamma_ref[...] + beta_ref[...]
    o_ref[...] = _gelu(ln)


def _update(nf_flat, s2, W2, b2, W3, b3, gamma, beta):
    rows = 5000
    grid = (BN // rows,)
    return pl.pallas_call(
        functools.partial(_update_body, rows=rows),
        grid=grid,
        in_specs=[
            pl.BlockSpec((rows, D), lambda i: (i, 0)),
            pl.BlockSpec((rows, D), lambda i: (i, 0)),
            pl.BlockSpec((M, M), lambda i: (0, 0)),
            pl.BlockSpec((1, M), lambda i: (0, 0)),
            pl.BlockSpec((D + M, D), lambda i: (0, 0)),
            pl.BlockSpec((1, D), lambda i: (0, 0)),
            pl.BlockSpec((1, D), lambda i: (0, 0)),
            pl.BlockSpec((1, D), lambda i: (0, 0)),
        ],
        out_specs=pl.BlockSpec((rows, D), lambda i: (i, 0)),
        out_shape=jax.ShapeDtypeStruct((BN, D), jnp.float32),
    )(nf_flat, s2, W2, b2, W3, b3, gamma, beta)


def kernel(node_features, neighbor_idx, weights, W1, b1, W2, b2, W3, b3,
           gamma, beta):
    nf_flat = node_features.reshape(BN, D)

    qpad = _encode(nf_flat, W1, b1.reshape(1, M))
    idx_t = jnp.swapaxes(neighbor_idx, 1, 2)
    w_t = jnp.swapaxes(weights, 1, 2)
    s_pad = _sc_gather_sum(qpad, idx_t, w_t)
    out = _update(nf_flat, s_pad, W2, b2.reshape(1, M),
                  W3, b3.reshape(1, D), gamma.reshape(1, D), beta.reshape(1, D))
    return out.reshape(B, N, D)
